# baseline (speedup 1.0000x reference)
import functools

import jax
import jax.numpy as jnp
from jax import lax
from jax.experimental import pallas as pl
from jax.experimental.pallas import tpu as pltpu
from jax.experimental.pallas import tpu_sc as plsc

_NC, _NS, _L = 2, 16, 16
_NEG_INF = float("-inf")


def _row_max(buf_ref, off, n, tmp):
    unroll = 8
    step = unroll * _L

    def body(j, accs):
        base = off + j * step
        accs = list(accs)
        for k in range(unroll):
            v = buf_ref[pl.ds(base + k * _L, _L)]
            accs[k % 4] = jnp.maximum(accs[k % 4], v)
        return tuple(accs)

    init = tuple(jnp.full((_L,), _NEG_INF, jnp.float32) for _ in range(4))
    a0, a1, a2, a3 = lax.fori_loop(0, n // step, body, init)
    u = jnp.maximum(jnp.maximum(a0, a1), jnp.maximum(a2, a3))
    for sh in (8, 4, 2, 1):
        tmp[pl.ds(0, _L)] = u
        u = jnp.maximum(u, tmp[pl.ds(sh, _L)])
    return u[0]


def _pack16(smem_ref, base):
    lanes = lax.iota(jnp.int32, _L)
    v = jnp.full((_L,), _NEG_INF, jnp.float32)
    for j in range(_L):
        v = jnp.where(lanes == j, smem_ref[base + j], v)
    return v


def _sc_body(C, D, N, xyz_hbm, pts_hbm, out_xyz, out_pts,
             b0, b1, b2, b3, xyz_buf, res_pts, res_xyz, tmp, sm_pts, sm_xyz,
             s0, s1, s2, s3, sx):
    wid = lax.axis_index("s") * _NC + lax.axis_index("c")

    bufs = (b0, b1, b2, b3)
    sems = (s0, s1, s2, s3)
    nbuf = 4
    tmp[pl.ds(_L, _L)] = jnp.full((_L,), _NEG_INF, jnp.float32)

    for c in range(C):
        pltpu.async_copy(xyz_hbm.at[wid, c], xyz_buf.at[pl.ds(c * N, N)], sx)

    for k in range(nbuf):
        pltpu.async_copy(pts_hbm.at[wid, k], bufs[k], sems[k])

    def chunk_loop(g, _):
        for k in range(nbuf):
            row = g * nbuf + k
            pltpu.make_async_copy(pts_hbm.at[0, 0], bufs[k], sems[k]).wait()
            sm_pts[row] = _row_max(bufs[k], 0, N, tmp)

            @pl.when(row + nbuf < D)
            def _():
                pltpu.async_copy(pts_hbm.at[wid, row + nbuf], bufs[k], sems[k])
        return 0

    lax.fori_loop(0, D // nbuf, chunk_loop, 0)

    for c in range(C):
        pltpu.make_async_copy(xyz_hbm.at[0, 0], xyz_buf.at[pl.ds(c * N, N)],
                              sx).wait()
    for c in range(C):
        sm_xyz[c] = _row_max(xyz_buf, c * N, N, tmp)
    for c in range(C, _L):
        sm_xyz[c] = 0.0

    for i in range(D // _L):
        res_pts[pl.ds(i * _L, _L)] = _pack16(sm_pts, i * _L)
    res_xyz[...] = _pack16(sm_xyz, 0)

    pltpu.sync_copy(res_pts, out_pts.at[pl.ds(wid * D, D)])
    pltpu.sync_copy(res_xyz, out_xyz.at[pl.ds(wid * _L, _L)])


def _sc_channel_max(xyz, points):
    B, C, N = xyz.shape
    D = points.shape[1]
    mesh = plsc.VectorSubcoreMesh(core_axis_name="c", subcore_axis_name="s")
    f = pl.kernel(
        functools.partial(_sc_body, C, D, N),
        out_type=[
            jax.ShapeDtypeStruct((B * _L,), jnp.float32),
            jax.ShapeDtypeStruct((B * D,), jnp.float32),
        ],
        mesh=mesh,
        scratch_types=[
            pltpu.VMEM((N,), jnp.float32),
            pltpu.VMEM((N,), jnp.float32),
            pltpu.VMEM((N,), jnp.float32),
            pltpu.VMEM((N,), jnp.float32),
            pltpu.VMEM((C * N,), jnp.float32),
            pltpu.VMEM((D,), jnp.float32),
            pltpu.VMEM((_L,), jnp.float32),
            pltpu.VMEM((2 * _L,), jnp.float32),
            pltpu.SMEM((D,), jnp.float32),
            pltpu.SMEM((_L,), jnp.float32),
            pltpu.SemaphoreType.DMA,
            pltpu.SemaphoreType.DMA,
            pltpu.SemaphoreType.DMA,
            pltpu.SemaphoreType.DMA,
            pltpu.SemaphoreType.DMA,
        ],
        compiler_params=pltpu.CompilerParams(use_tc_tiling_on_sc=False),
    )
    return f(xyz, points)


def kernel(xyz, points):
    B, C, N = xyz.shape
    D = points.shape[1]
    ox, op = _sc_channel_max(xyz, points)
    ox = ox.reshape(B, _L)[:, :C]
    op = op.reshape(B, D)
    new_points = jnp.concatenate([ox, op], axis=1)[:, :, None]
    new_xyz = jnp.zeros((B, C, 1), dtype=xyz.dtype)
    return (new_xyz, new_points)


# speedup vs baseline: 2.6551x; 2.6551x over previous
import functools

import jax
import jax.numpy as jnp
from jax import lax
from jax.experimental import pallas as pl
from jax.experimental.pallas import tpu as pltpu
from jax.experimental.pallas import tpu_sc as plsc

_NC, _NS, _L = 2, 16, 16
_NEG_INF = float("-inf")


def _row_max(buf_ref, n, tmp):
    unroll = 8
    step = unroll * _L

    def body(j, accs):
        base = j * step
        accs = list(accs)
        for k in range(unroll):
            v = buf_ref[0, pl.ds(base + k * _L, _L)]
            accs[k % 4] = jnp.maximum(accs[k % 4], v)
        return tuple(accs)

    init = tuple(jnp.full((_L,), _NEG_INF, jnp.float32) for _ in range(4))
    a0, a1, a2, a3 = lax.fori_loop(0, n // step, body, init)
    u = jnp.maximum(jnp.maximum(a0, a1), jnp.maximum(a2, a3))
    for sh in (8, 4, 2, 1):
        tmp[pl.ds(0, _L)] = u
        u = jnp.maximum(u, tmp[pl.ds(sh, _L)])
    return u[0]


def _pack16(smem_ref, base):
    lanes = lax.iota(jnp.int32, _L)
    v = jnp.full((_L,), _NEG_INF, jnp.float32)
    for j in range(_L):
        v = jnp.where(lanes == j, smem_ref[base + j], v)
    return v


def _sc_body(C, D, N, xyz_hbm, pts_hbm, out_xyz, out_pts,
             b0, b1, b2, b3, x0, x1, x2, res_pts, res_xyz, tmp,
             sm_pts, sm_xyz, s0, s1, s2, s3, sx):
    wid = lax.axis_index("s") * _NC + lax.axis_index("c")

    bufs = (b0, b1, b2, b3)
    xbufs = (x0, x1, x2)
    sems = (s0, s1, s2, s3)
    nbuf = 4

    tmp[pl.ds(_L, _L)] = jnp.full((_L,), _NEG_INF, jnp.float32)

    for c in range(C):
        pltpu.async_copy(xyz_hbm.at[wid, pl.ds(c, 1), :], xbufs[c], sx)

    for k in range(nbuf):
        pltpu.async_copy(pts_hbm.at[wid, pl.ds(k, 1), :], bufs[k], sems[k])

    def chunk_loop(g, _):
        for k in range(nbuf):
            row = g * nbuf + k
            pltpu.make_async_copy(pts_hbm.at[0, pl.ds(0, 1), :], bufs[k],
                                  sems[k]).wait()
            sm_pts[row] = _row_max(bufs[k], N, tmp)

            @pl.when(row + nbuf < D)
            def _():
                pltpu.async_copy(pts_hbm.at[wid, pl.ds(row + nbuf, 1), :],
                                 bufs[k], sems[k])
        return 0

    lax.fori_loop(0, D // nbuf, chunk_loop, 0)

    for c in range(C):
        pltpu.make_async_copy(xyz_hbm.at[0, pl.ds(0, 1), :], xbufs[c],
                              sx).wait()
    for c in range(C):
        sm_xyz[c] = _row_max(xbufs[c], N, tmp)
    for c in range(C, _L):
        sm_xyz[c] = 0.0

    for i in range(D // _L):
        res_pts[pl.ds(i * _L, _L)] = _pack16(sm_pts, i * _L)
    res_xyz[...] = _pack16(sm_xyz, 0)

    pltpu.sync_copy(res_pts, out_pts.at[pl.ds(wid * D, D)])
    pltpu.sync_copy(res_xyz, out_xyz.at[pl.ds(wid * _L, _L)])


def _sc_channel_max(xyz, points):
    B, C, N = xyz.shape
    D = points.shape[1]
    mesh = plsc.VectorSubcoreMesh(core_axis_name="c", subcore_axis_name="s")
    f = pl.kernel(
        functools.partial(_sc_body, C, D, N),
        out_type=[
            jax.ShapeDtypeStruct((B * _L,), jnp.float32),
            jax.ShapeDtypeStruct((B * D,), jnp.float32),
        ],
        mesh=mesh,
        scratch_types=[
            pltpu.VMEM((1, N), jnp.float32),
            pltpu.VMEM((1, N), jnp.float32),
            pltpu.VMEM((1, N), jnp.float32),
            pltpu.VMEM((1, N), jnp.float32),
            pltpu.VMEM((1, N), jnp.float32),
            pltpu.VMEM((1, N), jnp.float32),
            pltpu.VMEM((1, N), jnp.float32),
            pltpu.VMEM((D,), jnp.float32),
            pltpu.VMEM((_L,), jnp.float32),
            pltpu.VMEM((2 * _L,), jnp.float32),
            pltpu.SMEM((D,), jnp.float32),
            pltpu.SMEM((_L,), jnp.float32),
            pltpu.SemaphoreType.DMA,
            pltpu.SemaphoreType.DMA,
            pltpu.SemaphoreType.DMA,
            pltpu.SemaphoreType.DMA,
            pltpu.SemaphoreType.DMA,
        ],
        compiler_params=pltpu.CompilerParams(use_tc_tiling_on_sc=True),
    )
    return f(xyz, points)


def kernel(xyz, points):
    B, C, N = xyz.shape
    D = points.shape[1]
    ox, op = _sc_channel_max(xyz, points)
    ox = ox.reshape(B, _L)[:, :C]
    op = op.reshape(B, D)
    new_points = jnp.concatenate([ox, op], axis=1)[:, :, None]
    new_xyz = jnp.zeros((B, C, 1), dtype=xyz.dtype)
    return (new_xyz, new_points)
